# baseline (speedup 1.0000x reference)
import functools

import jax
import jax.numpy as jnp
from jax import lax
from jax.experimental import pallas as pl
from jax.experimental.pallas import tpu as pltpu
from jax.experimental.pallas import tpu_sc as plsc

N = 10000
D = 256
G = 64
E = 160000
NB = 32
R = 320
NR = NB * R
RT = R + 8
K = 128
C = 2000
EPAD = E + K

_mesh = plsc.VectorSubcoreMesh(core_axis_name="c", subcore_axis_name="s",
                               num_cores=2, num_subcores=16)

_i32 = jnp.int32
_f32 = jnp.float32


@functools.partial(
    pl.kernel,
    out_type=(
        jax.ShapeDtypeStruct((NB, EPAD), _i32),
        jax.ShapeDtypeStruct((NB, EPAD), _i32),
        jax.ShapeDtypeStruct((NB, EPAD), _i32),
        jax.ShapeDtypeStruct((NB, 16), _i32),
        jax.ShapeDtypeStruct((NB, 16), _i32),
        jax.ShapeDtypeStruct((NB, 16), _i32),
        jax.ShapeDtypeStruct((NR, 16), _f32),
        jax.ShapeDtypeStruct((NR, 16), _f32),
        jax.ShapeDtypeStruct((NR, 16), _f32),
    ),
    mesh=_mesh,
    scratch_types=[
        pltpu.VMEM((C,), _i32),
        pltpu.VMEM((C,), _i32),
        pltpu.VMEM((C + 304,), _i32),
        pltpu.VMEM((16,), _i32),
        pltpu.VMEM((K + 16,), _i32),
        pltpu.VMEM((RT, 16), _f32),
    ],
    compiler_params=pltpu.CompilerParams(needs_layout_passes=False, disable_bounds_checks=True),
)
def _sc_bucketize(es0, ed0, es1, ed1, es2, ed2, lst0, lst1, lst2,
                  cnt0, cnt1, cnt2, deg0, deg1, deg2, srcb, dstb, listb,
                  cvec, kbuf, acc16):
    c = lax.axis_index("c")
    s = lax.axis_index("s")
    w = c * 16 + s
    lo = w * R

    for es, ed, lst, cnt, deg in ((es0, ed0, lst0, cnt0, deg0),
                                  (es1, ed1, lst1, cnt1, deg1),
                                  (es2, ed2, lst2, cnt2, deg2)):
        def chunk_body(t, carry):
            pos, outb = carry
            pltpu.sync_copy(es.at[pl.ds(t * C, C)], srcb)
            pltpu.sync_copy(ed.at[pl.ds(t * C, C)], dstb)

            def vec_body(j, posv):
                sv = srcb[pl.ds(j * 16, 16)]
                dv = dstb[pl.ds(j * 16, 16)]
                dl = dv - lo
                m = (dl >= 0) & (dl < R)
                pk = (sv << 9) | jnp.where(m, dl, 0)
                kin = m.astype(_i32)
                excl = plsc.cumsum(kin) - kin
                idx = jnp.where(m, posv + excl, C + 303)
                plsc.store_scatter(listb, [idx], pk)
                return posv + plsc.all_reduce_population_count(m)
            posv = lax.fori_loop(0, C // 16, vec_body,
                                 jnp.zeros((16,), _i32) + pos, unroll=4)
            pos = posv[0]

            nblk = pos // K

            def flush(b, _):
                pltpu.sync_copy(listb.at[pl.ds(b * K, K)],
                                lst.at[w, pl.ds((outb + b) * K, K)])
                return 0
            lax.fori_loop(0, nblk, flush, 0)
            rs = nblk * K

            def shift(v, _):
                tmp = listb[pl.ds(rs + v * 16, 16)]
                listb[pl.ds(v * 16, 16)] = tmp
                return 0
            lax.fori_loop(0, K // 16, shift, 0)
            return pos - rs, outb + nblk

        pos, outb = lax.fori_loop(0, E // C, chunk_body,
                                  (jnp.asarray(0, _i32), jnp.asarray(0, _i32)))

        sent = jnp.full((16,), R, _i32)

        def pad(v, _):
            listb[pl.ds(pos + v * 16, 16)] = sent
            return 0
        lax.fori_loop(0, K // 16, pad, 0)
        nblk2 = (pos + K - 1) // K

        def flush2(b, _):
            pltpu.sync_copy(listb.at[pl.ds(b * K, K)],
                            lst.at[w, pl.ds((outb + b) * K, K)])
            return 0
        lax.fori_loop(0, nblk2, flush2, 0)
        total = (outb + nblk2) * K
        cvec[...] = jnp.zeros((16,), _i32) + total
        pltpu.sync_copy(cvec, cnt.at[w])

        def za16(i, _):
            acc16[i, pl.ds(0, 16)] = jnp.zeros((16,), _f32)
            return 0
        lax.fori_loop(0, RT, za16, 0)

        def count_chunk(t, _):
            pltpu.sync_copy(lst.at[w, pl.ds(t * K, K)], kbuf.at[pl.ds(0, K)])

            def one(k, _):
                dl = kbuf[pl.ds(k, 16)][0] & 511
                plsc.addupdate(acc16.at[dl, pl.ds(0, 16)],
                               jnp.ones((16,), _f32))
                return 0
            lax.fori_loop(0, K, one, 0)
            return 0
        lax.fori_loop(0, total // K, count_chunk, 0)
        pltpu.sync_copy(acc16.at[pl.ds(0, R)], deg.at[pl.ds(w * R, R)])


KB = 64


@functools.partial(
    pl.kernel,
    out_type=(
        jax.ShapeDtypeStruct((NR * D,), _f32),
        jax.ShapeDtypeStruct((NR * D,), _f32),
        jax.ShapeDtypeStruct((NR * D,), _f32),
    ),
    mesh=_mesh,
    scratch_types=[
        pltpu.VMEM((KB + 16,), _i32),
        pltpu.VMEM((KB + 16,), _i32),
        pltpu.VMEM((KB,), _i32),
        pltpu.VMEM((KB,), _i32),
        pltpu.VMEM((KB, D), _f32),
        pltpu.VMEM((KB, D), _f32),
        pltpu.VMEM((16,), _i32),
        pltpu.VMEM((RT * D,), _f32),
        pltpu.SMEM((KB,), _i32),
        pltpu.SMEM((KB,), _i32),
        pltpu.SemaphoreType.DMA,
        pltpu.SemaphoreType.DMA,
    ],
    compiler_params=pltpu.CompilerParams(needs_layout_passes=False, disable_bounds_checks=True),
)
def _sc_segsum3(h, hp, lst0, lst1, lst2, cnt0, cnt1, cnt2,
                o0, o1, o2, pkb0, pkb1, gsrc0, gsrc1, rows0, rows1,
                cntv, acc, sm0, sm1, sem0, sem1):
    c = lax.axis_index("c")
    s = lax.axis_index("s")
    w = c * 16 + s

    for tab, lst, cnt, out in ((hp, lst0, cnt0, o0),
                               (h, lst1, cnt1, o1),
                               (h, lst2, cnt2, o2)):
        def za(i, _):
            acc[pl.ds(i * 16, 16)] = jnp.zeros((16,), _f32)
            return 0
        lax.fori_loop(0, RT * 16, za, 0, unroll=4)

        pltpu.sync_copy(cnt.at[w], cntv)
        pairs = cntv[...][0] // (2 * KB)

        def mk_acc(sm, rows):
            def one(k, _):
                b = (sm[k] & 511) << 8
                for cc in range(16):
                    sl = pl.ds(b + cc * 16, 16)
                    acc[sl] = acc[sl] + rows[k, pl.ds(cc * 16, 16)]
                return 0
            return one

        def mk_ext(pkb, sm):
            def ext(k, _):
                sm[k] = pkb[pl.ds(k, 16)][0]
                return 0
            return ext

        def pair_chunk(t, _):
            e0 = t * 2 * KB
            pltpu.sync_copy(lst.at[w, pl.ds(e0, KB)], pkb0.at[pl.ds(0, KB)])

            def up0(j, _):
                gsrc0[pl.ds(j * 16, 16)] = pkb0[pl.ds(j * 16, 16)] >> 9
                return 0
            lax.fori_loop(0, KB // 16, up0, 0, unroll=4)
            d0 = pltpu.async_copy(tab.at[gsrc0], rows0, sem0)
            lax.fori_loop(0, KB, mk_ext(pkb0, sm0), 0, unroll=8)

            pltpu.sync_copy(lst.at[w, pl.ds(e0 + KB, KB)],
                            pkb1.at[pl.ds(0, KB)])

            def up1(j, _):
                gsrc1[pl.ds(j * 16, 16)] = pkb1[pl.ds(j * 16, 16)] >> 9
                return 0
            lax.fori_loop(0, KB // 16, up1, 0, unroll=4)
            d1 = pltpu.async_copy(tab.at[gsrc1], rows1, sem1)
            lax.fori_loop(0, KB, mk_ext(pkb1, sm1), 0, unroll=8)

            d0.wait()
            lax.fori_loop(0, KB, mk_acc(sm0, rows0), 0, unroll=4)
            d1.wait()
            lax.fori_loop(0, KB, mk_acc(sm1, rows1), 0, unroll=4)
            return 0
        lax.fori_loop(0, pairs, pair_chunk, 0)

        pltpu.sync_copy(acc.at[pl.ds(0, R * D)], out.at[pl.ds(w * R * D, R * D)])


def _prolog_body(x_ref, w_ref, b_ref, dga_ref, ca_ref, cb_ref,
                 h_ref, hp_ref, dinv_ref, ia_ref, ib_ref):
    h = jnp.maximum(jnp.dot(x_ref[...], w_ref[...],
                            preferred_element_type=_f32) + b_ref[...], 0.0)
    dinv = lax.rsqrt(dga_ref[...] + 1.0)
    h_ref[...] = h
    hp_ref[...] = h * dinv[:, None]
    dinv_ref[...] = dinv
    ia_ref[...] = 1.0 / jnp.maximum(ca_ref[...], 1.0)
    ib_ref[...] = 1.0 / jnp.maximum(cb_ref[...], 1.0)


def _tc_prolog(x, w, b, dga, ca, cb):
    blk = 256
    grid = pl.cdiv(N, blk)
    vspec = pl.BlockSpec((blk,), lambda i: (i,))
    return pl.pallas_call(
        _prolog_body,
        grid=(grid,),
        in_specs=[
            pl.BlockSpec((blk, D), lambda i: (i, 0)),
            pl.BlockSpec((D, D), lambda i: (0, 0)),
            pl.BlockSpec((D,), lambda i: (0,)),
            vspec, vspec, vspec,
        ],
        out_specs=[
            pl.BlockSpec((blk, D), lambda i: (i, 0)),
            pl.BlockSpec((blk, D), lambda i: (i, 0)),
            vspec, vspec, vspec,
        ],
        out_shape=[
            jax.ShapeDtypeStruct((N, D), _f32),
            jax.ShapeDtypeStruct((N, D), _f32),
            jax.ShapeDtypeStruct((N,), _f32),
            jax.ShapeDtypeStruct((N,), _f32),
            jax.ShapeDtypeStruct((N,), _f32),
        ],
    )(x, w, b, dga, ca, cb)


def _layer_body(ma_ref, md_ref, mc_ref, h_ref, hp_ref, dinv_ref, ia_ref,
                ib_ref, u_ref, bias_ref, g_ref, lb_ref, hn_ref, hpn_ref):
    dinv = dinv_ref[...][:, None]
    a = (ma_ref[...] + hp_ref[...]) * dinv
    b = md_ref[...] * ia_ref[...][:, None]
    cc = mc_ref[...] * ib_ref[...][:, None]
    x = jnp.concatenate([a, b, cc, h_ref[...]], axis=1)
    out = jnp.dot(x, u_ref[...], preferred_element_type=_f32) + bias_ref[...]
    out = jnp.maximum(out, 0.0)
    mu = jnp.mean(out, axis=1, keepdims=True)
    var = jnp.mean((out - mu) ** 2, axis=1, keepdims=True)
    hn = (out - mu) * lax.rsqrt(var + 1e-5) * g_ref[...] + lb_ref[...]
    hn_ref[...] = hn
    hpn_ref[...] = hn * dinv


def _tc_layer(ma, md, mc, h, hp, dinv, ia, ib, u, bias, g, lb):
    blk = 256
    grid = pl.cdiv(N, blk)
    mspec = pl.BlockSpec((blk, D), lambda i: (i, 0))
    vspec = pl.BlockSpec((blk,), lambda i: (i,))
    return pl.pallas_call(
        _layer_body,
        grid=(grid,),
        in_specs=[
            mspec, mspec, mspec, mspec, mspec,
            vspec, vspec, vspec,
            pl.BlockSpec((4 * D, D), lambda i: (0, 0)),
            pl.BlockSpec((D,), lambda i: (0,)),
            pl.BlockSpec((D,), lambda i: (0,)),
            pl.BlockSpec((D,), lambda i: (0,)),
        ],
        out_specs=[mspec, mspec],
        out_shape=[
            jax.ShapeDtypeStruct((N, D), _f32),
            jax.ShapeDtypeStruct((N, D), _f32),
        ],
    )(ma, md, mc, h, hp, dinv, ia, ib, u, bias, g, lb)


def _epi_body(h_ref, bn_ref, attw_ref, outw_ref, outb_ref, o_ref):
    h = h_ref[...]
    logits = jnp.dot(h, attw_ref[...], preferred_element_type=_f32)[:, 0]
    p = jnp.exp(logits - jnp.max(logits))
    p = p / jnp.sum(p)
    wtd = h * p[:, None]
    oh = (bn_ref[...][:, None] ==
          lax.broadcasted_iota(_i32, (N, G), 1)).astype(_f32)
    gr = lax.dot_general(oh, wtd, (((0,), (0,)), ((), ())),
                         preferred_element_type=_f32)
    gc = lax.dot_general(oh, p[:, None], (((0,), (0,)), ((), ())),
                         preferred_element_type=_f32)
    gr = gr / jnp.maximum(gc, 1e-6)
    o_ref[...] = jnp.dot(gr, outw_ref[...],
                         preferred_element_type=_f32) + outb_ref[...]


def _tc_epilogue(h, bn, attw, outw, outb):
    return pl.pallas_call(
        _epi_body,
        out_shape=jax.ShapeDtypeStruct((G, D), _f32),
    )(h, bn, attw, outw, outb)


def kernel(x_node, edge_index_ast, edge_index_df, edge_index_cf, batch_node,
           proj_W, proj_b, gcn_W, gcn_b, sgA_Wl, sgA_bl, sgA_Wr,
           sgB_Wl, sgB_bl, sgB_Wr, ln_g, ln_b, att_W, att_b, out_W, out_b):
    (lst0, lst1, lst2, cnt0, cnt1, cnt2,
     deg0, deg1, deg2) = _sc_bucketize(
         edge_index_ast[0], edge_index_ast[1],
         edge_index_df[0], edge_index_df[1],
         edge_index_cf[0], edge_index_cf[1])
    dga = deg0[:N, 0]
    ca = deg1[:N, 0]
    cb = deg2[:N, 0]
    import os as _os
    if _os.environ.get("_BISECT") == "A":
        return (dga[:G, None] + ca[:G, None] + cb[:G, None]
                + jnp.zeros((G, D), _f32)
                + lst0[0, 0] + cnt0[0, 0])

    h, hp, dinv, ia, ib = _tc_prolog(x_node, proj_W, proj_b, dga, ca, cb)

    for l in range(gcn_W.shape[0]):
        u = jnp.concatenate(
            [gcn_W[l], sgA_Wl[l], sgB_Wl[l], sgA_Wr[l] + sgB_Wr[l]], axis=0)
        bias = gcn_b[l] + sgA_bl[l] + sgB_bl[l]
        ma, md, mc = _sc_segsum3(h, hp, lst0, lst1, lst2, cnt0, cnt1, cnt2)
        ma = ma.reshape(NR, D)[:N]
        md = md.reshape(NR, D)[:N]
        mc = mc.reshape(NR, D)[:N]
        h, hp = _tc_layer(ma, md, mc, h, hp, dinv, ia, ib,
                          u, bias, ln_g, ln_b)

    return _tc_epilogue(h, batch_node, att_W, out_W, out_b)


# speedup vs baseline: 1.1429x; 1.1429x over previous
import functools

import jax
import jax.numpy as jnp
from jax import lax
from jax.experimental import pallas as pl
from jax.experimental.pallas import tpu as pltpu
from jax.experimental.pallas import tpu_sc as plsc

N = 10000
D = 256
G = 64
E = 160000
NB = 32
R = 320
NR = NB * R
RT = R + 8
K = 128
C = 2000
EPAD = E + K

_mesh = plsc.VectorSubcoreMesh(core_axis_name="c", subcore_axis_name="s",
                               num_cores=2, num_subcores=16)

_i32 = jnp.int32
_f32 = jnp.float32


@functools.partial(
    pl.kernel,
    out_type=(
        jax.ShapeDtypeStruct((NB, EPAD), _i32),
        jax.ShapeDtypeStruct((NB, EPAD), _i32),
        jax.ShapeDtypeStruct((NB, EPAD), _i32),
        jax.ShapeDtypeStruct((NB, 16), _i32),
        jax.ShapeDtypeStruct((NB, 16), _i32),
        jax.ShapeDtypeStruct((NB, 16), _i32),
        jax.ShapeDtypeStruct((NR, 16), _f32),
        jax.ShapeDtypeStruct((NR, 16), _f32),
        jax.ShapeDtypeStruct((NR, 16), _f32),
    ),
    mesh=_mesh,
    scratch_types=[
        pltpu.VMEM((C,), _i32),
        pltpu.VMEM((C,), _i32),
        pltpu.VMEM((C + 304,), _i32),
        pltpu.VMEM((16,), _i32),
        pltpu.VMEM((K + 16,), _i32),
        pltpu.VMEM((RT, 16), _f32),
    ],
    compiler_params=pltpu.CompilerParams(needs_layout_passes=False, disable_bounds_checks=True),
)
def _sc_bucketize(es0, ed0, es1, ed1, es2, ed2, lst0, lst1, lst2,
                  cnt0, cnt1, cnt2, deg0, deg1, deg2, srcb, dstb, listb,
                  cvec, kbuf, acc16):
    c = lax.axis_index("c")
    s = lax.axis_index("s")
    w = c * 16 + s
    lo = w * R

    for es, ed, lst, cnt, deg in ((es0, ed0, lst0, cnt0, deg0),
                                  (es1, ed1, lst1, cnt1, deg1),
                                  (es2, ed2, lst2, cnt2, deg2)):
        def chunk_body(t, carry):
            pos, outb = carry
            pltpu.sync_copy(es.at[pl.ds(t * C, C)], srcb)
            pltpu.sync_copy(ed.at[pl.ds(t * C, C)], dstb)

            def vec_body(j, posv):
                sv = srcb[pl.ds(j * 16, 16)]
                dv = dstb[pl.ds(j * 16, 16)]
                dl = dv - lo
                m = (dl >= 0) & (dl < R)
                pk = (sv << 9) | jnp.where(m, dl, 0)
                kin = m.astype(_i32)
                excl = plsc.cumsum(kin) - kin
                idx = jnp.where(m, posv + excl, C + 303)
                plsc.store_scatter(listb, [idx], pk)
                return posv + plsc.all_reduce_population_count(m)
            posv = lax.fori_loop(0, C // 16, vec_body,
                                 jnp.zeros((16,), _i32) + pos, unroll=4)
            pos = posv[0]

            nblk = pos // K

            def flush(b, _):
                pltpu.sync_copy(listb.at[pl.ds(b * K, K)],
                                lst.at[w, pl.ds((outb + b) * K, K)])
                return 0
            lax.fori_loop(0, nblk, flush, 0)
            rs = nblk * K

            def shift(v, _):
                tmp = listb[pl.ds(rs + v * 16, 16)]
                listb[pl.ds(v * 16, 16)] = tmp
                return 0
            lax.fori_loop(0, K // 16, shift, 0)
            return pos - rs, outb + nblk

        pos, outb = lax.fori_loop(0, E // C, chunk_body,
                                  (jnp.asarray(0, _i32), jnp.asarray(0, _i32)))

        sent = jnp.full((16,), R, _i32)

        def pad(v, _):
            listb[pl.ds(pos + v * 16, 16)] = sent
            return 0
        lax.fori_loop(0, K // 16, pad, 0)
        nblk2 = (pos + K - 1) // K

        def flush2(b, _):
            pltpu.sync_copy(listb.at[pl.ds(b * K, K)],
                            lst.at[w, pl.ds((outb + b) * K, K)])
            return 0
        lax.fori_loop(0, nblk2, flush2, 0)
        total = (outb + nblk2) * K
        cvec[...] = jnp.zeros((16,), _i32) + total
        pltpu.sync_copy(cvec, cnt.at[w])

        def za16(i, _):
            acc16[i, pl.ds(0, 16)] = jnp.zeros((16,), _f32)
            return 0
        lax.fori_loop(0, RT, za16, 0)

        def count_chunk(t, _):
            pltpu.sync_copy(lst.at[w, pl.ds(t * K, K)], kbuf.at[pl.ds(0, K)])

            def one(k, _):
                dl = kbuf[pl.ds(k, 16)][0] & 511
                plsc.addupdate(acc16.at[dl, pl.ds(0, 16)],
                               jnp.ones((16,), _f32))
                return 0
            lax.fori_loop(0, K, one, 0)
            return 0
        lax.fori_loop(0, total // K, count_chunk, 0)
        pltpu.sync_copy(acc16.at[pl.ds(0, R)], deg.at[pl.ds(w * R, R)])


KB = 64


@functools.partial(
    pl.kernel,
    out_type=tuple(
        jax.ShapeDtypeStruct((NR * (D // 4),), _f32)
        for _ in range(12)
    ),
    mesh=_mesh,
    scratch_types=[
        pltpu.VMEM((KB + 16,), _i32),
        pltpu.VMEM((KB + 16,), _i32),
        pltpu.VMEM((KB,), _i32),
        pltpu.VMEM((KB,), _i32),
        pltpu.VMEM((KB, D), _f32),
        pltpu.VMEM((KB, D), _f32),
        pltpu.VMEM((16,), _i32),
        pltpu.VMEM((RT * (D // 4),), _f32),
        pltpu.VMEM((RT * (D // 4),), _f32),
        pltpu.VMEM((RT * (D // 4),), _f32),
        pltpu.VMEM((RT * (D // 4),), _f32),
        pltpu.SMEM((KB,), _i32),
        pltpu.SMEM((KB,), _i32),
        pltpu.SemaphoreType.DMA,
        pltpu.SemaphoreType.DMA,
    ],
    compiler_params=pltpu.CompilerParams(needs_layout_passes=False, disable_bounds_checks=True),
)
def _sc_segsum3(h, hp, lst0, lst1, lst2, cnt0, cnt1, cnt2,
                oa0, oa1, oa2, oa3, ob0, ob1, ob2, ob3, oc0, oc1, oc2, oc3,
                pkb0, pkb1, gsrc0, gsrc1, rows0, rows1,
                cntv, acc0, acc1, acc2, acc3, sm0, sm1, sem0, sem1):
    c = lax.axis_index("c")
    s = lax.axis_index("s")
    w = c * 16 + s
    accs = (acc0, acc1, acc2, acc3)
    DQ = D // 4

    for tab, lst, cnt, outs in ((hp, lst0, cnt0, (oa0, oa1, oa2, oa3)),
                                (h, lst1, cnt1, (ob0, ob1, ob2, ob3)),
                                (h, lst2, cnt2, (oc0, oc1, oc2, oc3))):
        for acc in accs:
            def za(i, _):
                acc[pl.ds(i * 16, 16)] = jnp.zeros((16,), _f32)
                return 0
            lax.fori_loop(0, RT * DQ // 16, za, 0, unroll=4)

        pltpu.sync_copy(cnt.at[w], cntv)
        pairs = cntv[...][0] // (2 * KB)

        def mk_acc(sm, rows):
            def one(k, _):
                b = (sm[k] & 511) << 6
                for q in range(4):
                    for cc in range(4):
                        plsc.addupdate(
                            accs[q].at[pl.ds(b + cc * 16, 16)],
                            rows[k, pl.ds(q * 64 + cc * 16, 16)])
                return 0
            return one

        def mk_ext(pkb, sm):
            def ext(k, _):
                sm[k] = pkb[pl.ds(k, 16)][0]
                return 0
            return ext

        def pair_chunk(t, _):
            e0 = t * 2 * KB
            pltpu.sync_copy(lst.at[w, pl.ds(e0, KB)], pkb0.at[pl.ds(0, KB)])

            def up0(j, _):
                gsrc0[pl.ds(j * 16, 16)] = pkb0[pl.ds(j * 16, 16)] >> 9
                return 0
            lax.fori_loop(0, KB // 16, up0, 0, unroll=4)
            d0 = pltpu.async_copy(tab.at[gsrc0], rows0, sem0)
            lax.fori_loop(0, KB, mk_ext(pkb0, sm0), 0, unroll=8)

            pltpu.sync_copy(lst.at[w, pl.ds(e0 + KB, KB)],
                            pkb1.at[pl.ds(0, KB)])

            def up1(j, _):
                gsrc1[pl.ds(j * 16, 16)] = pkb1[pl.ds(j * 16, 16)] >> 9
                return 0
            lax.fori_loop(0, KB // 16, up1, 0, unroll=4)
            d1 = pltpu.async_copy(tab.at[gsrc1], rows1, sem1)
            lax.fori_loop(0, KB, mk_ext(pkb1, sm1), 0, unroll=8)

            d0.wait()
            lax.fori_loop(0, KB, mk_acc(sm0, rows0), 0, unroll=4)
            d1.wait()
            lax.fori_loop(0, KB, mk_acc(sm1, rows1), 0, unroll=4)
            return 0
        lax.fori_loop(0, pairs, pair_chunk, 0)

        for q in range(4):
            pltpu.sync_copy(accs[q].at[pl.ds(0, R * DQ)],
                            outs[q].at[pl.ds(w * R * DQ, R * DQ)])


def _prolog_body(x_ref, w_ref, b_ref, dga_ref, ca_ref, cb_ref,
                 h_ref, hp_ref, dinv_ref, ia_ref, ib_ref):
    h = jnp.maximum(jnp.dot(x_ref[...], w_ref[...],
                            preferred_element_type=_f32) + b_ref[...], 0.0)
    dinv = lax.rsqrt(dga_ref[...] + 1.0)
    h_ref[...] = h
    hp_ref[...] = h * dinv[:, None]
    dinv_ref[...] = dinv
    ia_ref[...] = 1.0 / jnp.maximum(ca_ref[...], 1.0)
    ib_ref[...] = 1.0 / jnp.maximum(cb_ref[...], 1.0)


def _tc_prolog(x, w, b, dga, ca, cb):
    blk = 256
    grid = pl.cdiv(N, blk)
    vspec = pl.BlockSpec((blk,), lambda i: (i,))
    return pl.pallas_call(
        _prolog_body,
        grid=(grid,),
        in_specs=[
            pl.BlockSpec((blk, D), lambda i: (i, 0)),
            pl.BlockSpec((D, D), lambda i: (0, 0)),
            pl.BlockSpec((D,), lambda i: (0,)),
            vspec, vspec, vspec,
        ],
        out_specs=[
            pl.BlockSpec((blk, D), lambda i: (i, 0)),
            pl.BlockSpec((blk, D), lambda i: (i, 0)),
            vspec, vspec, vspec,
        ],
        out_shape=[
            jax.ShapeDtypeStruct((N, D), _f32),
            jax.ShapeDtypeStruct((N, D), _f32),
            jax.ShapeDtypeStruct((N,), _f32),
            jax.ShapeDtypeStruct((N,), _f32),
            jax.ShapeDtypeStruct((N,), _f32),
        ],
    )(x, w, b, dga, ca, cb)


def _layer_body(ma_ref, md_ref, mc_ref, h_ref, hp_ref, dinv_ref, ia_ref,
                ib_ref, u_ref, bias_ref, g_ref, lb_ref, hn_ref, hpn_ref):
    dinv = dinv_ref[...][:, None]
    a = (ma_ref[...] + hp_ref[...]) * dinv
    b = md_ref[...] * ia_ref[...][:, None]
    cc = mc_ref[...] * ib_ref[...][:, None]
    x = jnp.concatenate([a, b, cc, h_ref[...]], axis=1)
    out = jnp.dot(x, u_ref[...], preferred_element_type=_f32) + bias_ref[...]
    out = jnp.maximum(out, 0.0)
    mu = jnp.mean(out, axis=1, keepdims=True)
    var = jnp.mean((out - mu) ** 2, axis=1, keepdims=True)
    hn = (out - mu) * lax.rsqrt(var + 1e-5) * g_ref[...] + lb_ref[...]
    hn_ref[...] = hn
    hpn_ref[...] = hn * dinv


def _tc_layer(ma, md, mc, h, hp, dinv, ia, ib, u, bias, g, lb):
    blk = 256
    grid = pl.cdiv(N, blk)
    mspec = pl.BlockSpec((blk, D), lambda i: (i, 0))
    vspec = pl.BlockSpec((blk,), lambda i: (i,))
    return pl.pallas_call(
        _layer_body,
        grid=(grid,),
        in_specs=[
            mspec, mspec, mspec, mspec, mspec,
            vspec, vspec, vspec,
            pl.BlockSpec((4 * D, D), lambda i: (0, 0)),
            pl.BlockSpec((D,), lambda i: (0,)),
            pl.BlockSpec((D,), lambda i: (0,)),
            pl.BlockSpec((D,), lambda i: (0,)),
        ],
        out_specs=[mspec, mspec],
        out_shape=[
            jax.ShapeDtypeStruct((N, D), _f32),
            jax.ShapeDtypeStruct((N, D), _f32),
        ],
    )(ma, md, mc, h, hp, dinv, ia, ib, u, bias, g, lb)


def _epi_body(h_ref, bn_ref, attw_ref, outw_ref, outb_ref, o_ref):
    h = h_ref[...]
    logits = jnp.dot(h, attw_ref[...], preferred_element_type=_f32)[:, 0]
    p = jnp.exp(logits - jnp.max(logits))
    p = p / jnp.sum(p)
    wtd = h * p[:, None]
    oh = (bn_ref[...][:, None] ==
          lax.broadcasted_iota(_i32, (N, G), 1)).astype(_f32)
    gr = lax.dot_general(oh, wtd, (((0,), (0,)), ((), ())),
                         preferred_element_type=_f32)
    gc = lax.dot_general(oh, p[:, None], (((0,), (0,)), ((), ())),
                         preferred_element_type=_f32)
    gr = gr / jnp.maximum(gc, 1e-6)
    o_ref[...] = jnp.dot(gr, outw_ref[...],
                         preferred_element_type=_f32) + outb_ref[...]


def _tc_epilogue(h, bn, attw, outw, outb):
    return pl.pallas_call(
        _epi_body,
        out_shape=jax.ShapeDtypeStruct((G, D), _f32),
    )(h, bn, attw, outw, outb)


def kernel(x_node, edge_index_ast, edge_index_df, edge_index_cf, batch_node,
           proj_W, proj_b, gcn_W, gcn_b, sgA_Wl, sgA_bl, sgA_Wr,
           sgB_Wl, sgB_bl, sgB_Wr, ln_g, ln_b, att_W, att_b, out_W, out_b):
    (lst0, lst1, lst2, cnt0, cnt1, cnt2,
     deg0, deg1, deg2) = _sc_bucketize(
         edge_index_ast[0], edge_index_ast[1],
         edge_index_df[0], edge_index_df[1],
         edge_index_cf[0], edge_index_cf[1])
    dga = deg0[:N, 0]
    ca = deg1[:N, 0]
    cb = deg2[:N, 0]
    import os as _os
    if _os.environ.get("_BISECT") == "A":
        return (dga[:G, None] + ca[:G, None] + cb[:G, None]
                + jnp.zeros((G, D), _f32)
                + lst0[0, 0] + cnt0[0, 0])

    h, hp, dinv, ia, ib = _tc_prolog(x_node, proj_W, proj_b, dga, ca, cb)

    for l in range(gcn_W.shape[0]):
        u = jnp.concatenate(
            [gcn_W[l], sgA_Wl[l], sgB_Wl[l], sgA_Wr[l] + sgB_Wr[l]], axis=0)
        bias = gcn_b[l] + sgA_bl[l] + sgB_bl[l]
        parts = _sc_segsum3(h, hp, lst0, lst1, lst2, cnt0, cnt1, cnt2)
        DQ = D // 4
        ma, md, mc = (
            jnp.concatenate([p.reshape(NR, DQ) for p in parts[i:i + 4]],
                            axis=1)[:N]
            for i in (0, 4, 8))
        h, hp = _tc_layer(ma, md, mc, h, hp, dinv, ia, ib,
                          u, bias, ln_g, ln_b)

    return _tc_epilogue(h, batch_node, att_W, out_W, out_b)


# speedup vs baseline: 1.2629x; 1.1051x over previous
import functools

import jax
import jax.numpy as jnp
from jax import lax
from jax.experimental import pallas as pl
from jax.experimental.pallas import tpu as pltpu
from jax.experimental.pallas import tpu_sc as plsc

N = 10000
D = 256
G = 64
E = 160000
NB = 32
R = 320
NR = NB * R
RT = R + 8
K = 128
C = 2000
EPAD = E + K

_mesh = plsc.VectorSubcoreMesh(core_axis_name="c", subcore_axis_name="s",
                               num_cores=2, num_subcores=16)

_i32 = jnp.int32
_f32 = jnp.float32


@functools.partial(
    pl.kernel,
    out_type=(
        jax.ShapeDtypeStruct((NB, EPAD), _i32),
        jax.ShapeDtypeStruct((NB, EPAD), _i32),
        jax.ShapeDtypeStruct((NB, EPAD), _i32),
        jax.ShapeDtypeStruct((NB, 16), _i32),
        jax.ShapeDtypeStruct((NB, 16), _i32),
        jax.ShapeDtypeStruct((NB, 16), _i32),
        jax.ShapeDtypeStruct((NR, 16), _f32),
        jax.ShapeDtypeStruct((NR, 16), _f32),
        jax.ShapeDtypeStruct((NR, 16), _f32),
    ),
    mesh=_mesh,
    scratch_types=[
        pltpu.VMEM((C,), _i32),
        pltpu.VMEM((C,), _i32),
        pltpu.VMEM((C,), _i32),
        pltpu.VMEM((C,), _i32),
        pltpu.VMEM((C + 304,), _i32),
        pltpu.VMEM((16,), _i32),
        pltpu.VMEM((K + 16,), _i32),
        pltpu.VMEM((RT, 16), _f32),
        pltpu.SMEM((K,), _i32),
        pltpu.SemaphoreType.DMA,
        pltpu.SemaphoreType.DMA,
        pltpu.SemaphoreType.DMA,
        pltpu.SemaphoreType.DMA,
    ],
    compiler_params=pltpu.CompilerParams(needs_layout_passes=False, disable_bounds_checks=True),
)
def _sc_bucketize(es0, ed0, es1, ed1, es2, ed2, lst0, lst1, lst2,
                  cnt0, cnt1, cnt2, deg0, deg1, deg2, srcb0, dstb0,
                  srcb1, dstb1, listb, cvec, kbuf, acc16, smk,
                  sema0, semb0, sema1, semb1):
    c = lax.axis_index("c")
    s = lax.axis_index("s")
    w = c * 16 + s
    lo = w * R

    for es, ed, lst, cnt, deg in ((es0, ed0, lst0, cnt0, deg0),
                                  (es1, ed1, lst1, cnt1, deg1),
                                  (es2, ed2, lst2, cnt2, deg2)):
        def mk_scan(srcb, dstb):
            def vec_body(j, posv):
                sv = srcb[pl.ds(j * 16, 16)]
                dv = dstb[pl.ds(j * 16, 16)]
                dl = dv - lo
                m = (dl >= 0) & (dl < R)
                pk = (sv << 9) | jnp.where(m, dl, 0)
                kin = m.astype(_i32)
                excl = plsc.cumsum(kin) - kin
                idx = jnp.where(m, posv + excl, C + 303)
                plsc.store_scatter(listb, [idx], pk)
                return posv + plsc.all_reduce_population_count(m)
            return vec_body

        def flush_fn(pos, outb):
            nblk = pos // K

            def flush(b, _):
                pltpu.sync_copy(listb.at[pl.ds(b * K, K)],
                                lst.at[w, pl.ds((outb + b) * K, K)])
                return 0
            lax.fori_loop(0, nblk, flush, 0)
            rs = nblk * K

            def shift(v, _):
                tmp = listb[pl.ds(rs + v * 16, 16)]
                listb[pl.ds(v * 16, 16)] = tmp
                return 0
            lax.fori_loop(0, K // 16, shift, 0)
            return pos - rs, outb + nblk

        def pair_body(t, carry):
            pos, outb = carry
            e0 = t * 2 * C
            d0a = pltpu.async_copy(es.at[pl.ds(e0, C)], srcb0, sema0)
            d0b = pltpu.async_copy(ed.at[pl.ds(e0, C)], dstb0, semb0)
            d1a = pltpu.async_copy(es.at[pl.ds(e0 + C, C)], srcb1, sema1)
            d1b = pltpu.async_copy(ed.at[pl.ds(e0 + C, C)], dstb1, semb1)
            d0a.wait()
            d0b.wait()
            posv = lax.fori_loop(0, C // 16, mk_scan(srcb0, dstb0),
                                 jnp.zeros((16,), _i32) + pos, unroll=4)
            pos, outb = flush_fn(posv[0], outb)
            d1a.wait()
            d1b.wait()
            posv = lax.fori_loop(0, C // 16, mk_scan(srcb1, dstb1),
                                 jnp.zeros((16,), _i32) + pos, unroll=4)
            pos, outb = flush_fn(posv[0], outb)
            return pos, outb

        pos, outb = lax.fori_loop(0, E // C // 2, pair_body,
                                  (jnp.asarray(0, _i32), jnp.asarray(0, _i32)))

        sent = jnp.full((16,), R, _i32)

        def pad(v, _):
            listb[pl.ds(pos + v * 16, 16)] = sent
            return 0
        lax.fori_loop(0, K // 16, pad, 0)
        nblk2 = (pos + K - 1) // K

        def flush2(b, _):
            pltpu.sync_copy(listb.at[pl.ds(b * K, K)],
                            lst.at[w, pl.ds((outb + b) * K, K)])
            return 0
        lax.fori_loop(0, nblk2, flush2, 0)
        total = (outb + nblk2) * K
        cvec[...] = jnp.zeros((16,), _i32) + total
        pltpu.sync_copy(cvec, cnt.at[w])

        def za16(i, _):
            acc16[i, pl.ds(0, 16)] = jnp.zeros((16,), _f32)
            return 0
        lax.fori_loop(0, RT, za16, 0)

        def count_chunk(t, _):
            pltpu.sync_copy(lst.at[w, pl.ds(t * K, K)], kbuf.at[pl.ds(0, K)])

            def ext(k, _):
                smk[k] = kbuf[pl.ds(k, 16)][0]
                return 0
            lax.fori_loop(0, K, ext, 0, unroll=8)

            def one(k, _):
                plsc.addupdate(acc16.at[smk[k] & 511, pl.ds(0, 16)],
                               jnp.ones((16,), _f32))
                return 0
            lax.fori_loop(0, K, one, 0, unroll=4)
            return 0
        lax.fori_loop(0, total // K, count_chunk, 0)
        pltpu.sync_copy(acc16.at[pl.ds(0, R)], deg.at[pl.ds(w * R, R)])


KB = 64


@functools.partial(
    pl.kernel,
    out_type=(
        jax.ShapeDtypeStruct((NR * D,), _f32),
        jax.ShapeDtypeStruct((NR * D,), _f32),
        jax.ShapeDtypeStruct((NR * D,), _f32),
    ),
    mesh=_mesh,
    scratch_types=[
        pltpu.VMEM((KB + 16,), _i32),
        pltpu.VMEM((KB + 16,), _i32),
        pltpu.VMEM((KB,), _i32),
        pltpu.VMEM((KB,), _i32),
        pltpu.VMEM((KB, D), _f32),
        pltpu.VMEM((KB, D), _f32),
        pltpu.VMEM((16,), _i32),
        pltpu.VMEM((RT * D,), _f32),
        pltpu.SMEM((KB,), _i32),
        pltpu.SMEM((KB,), _i32),
        pltpu.SemaphoreType.DMA,
        pltpu.SemaphoreType.DMA,
    ],
    compiler_params=pltpu.CompilerParams(needs_layout_passes=False, disable_bounds_checks=True),
)
def _sc_segsum3(h, hp, lst0, lst1, lst2, cnt0, cnt1, cnt2,
                o0, o1, o2, pkb0, pkb1, gsrc0, gsrc1, rows0, rows1,
                cntv, acc, sm0, sm1, sem0, sem1):
    c = lax.axis_index("c")
    s = lax.axis_index("s")
    w = c * 16 + s

    for tab, lst, cnt, out in ((hp, lst0, cnt0, o0),
                               (h, lst1, cnt1, o1),
                               (h, lst2, cnt2, o2)):
        def za(i, _):
            acc[pl.ds(i * 16, 16)] = jnp.zeros((16,), _f32)
            return 0
        lax.fori_loop(0, RT * D // 16, za, 0, unroll=4)

        pltpu.sync_copy(cnt.at[w], cntv)
        pairs = cntv[...][0] // (2 * KB)

        def mk_acc(sm, rows):
            def one(k, _):
                b = (sm[k] & 511) << 8
                for cc in range(16):
                    plsc.addupdate(acc.at[pl.ds(b + cc * 16, 16)],
                                   rows[k, pl.ds(cc * 16, 16)])
                return 0
            return one

        def mk_ext(pkb, sm):
            def ext(k, _):
                sm[k] = pkb[pl.ds(k, 16)][0]
                return 0
            return ext

        def pair_chunk(t, _):
            e0 = t * 2 * KB
            pltpu.sync_copy(lst.at[w, pl.ds(e0, KB)], pkb0.at[pl.ds(0, KB)])

            def up0(j, _):
                gsrc0[pl.ds(j * 16, 16)] = pkb0[pl.ds(j * 16, 16)] >> 9
                return 0
            lax.fori_loop(0, KB // 16, up0, 0, unroll=4)
            d0 = pltpu.async_copy(tab.at[gsrc0], rows0, sem0)
            lax.fori_loop(0, KB, mk_ext(pkb0, sm0), 0, unroll=8)

            pltpu.sync_copy(lst.at[w, pl.ds(e0 + KB, KB)],
                            pkb1.at[pl.ds(0, KB)])

            def up1(j, _):
                gsrc1[pl.ds(j * 16, 16)] = pkb1[pl.ds(j * 16, 16)] >> 9
                return 0
            lax.fori_loop(0, KB // 16, up1, 0, unroll=4)
            d1 = pltpu.async_copy(tab.at[gsrc1], rows1, sem1)
            lax.fori_loop(0, KB, mk_ext(pkb1, sm1), 0, unroll=8)

            d0.wait()
            lax.fori_loop(0, KB, mk_acc(sm0, rows0), 0, unroll=4)
            d1.wait()
            lax.fori_loop(0, KB, mk_acc(sm1, rows1), 0, unroll=4)
            return 0
        lax.fori_loop(0, pairs, pair_chunk, 0)

        pltpu.sync_copy(acc.at[pl.ds(0, R * D)],
                        out.at[pl.ds(w * R * D, R * D)])


def _prolog_body(x_ref, w_ref, b_ref, dga_ref, ca_ref, cb_ref,
                 h_ref, hp_ref, dinv_ref, ia_ref, ib_ref):
    h = jnp.maximum(jnp.dot(x_ref[...], w_ref[...],
                            preferred_element_type=_f32) + b_ref[...], 0.0)
    dinv = lax.rsqrt(dga_ref[...] + 1.0)
    h_ref[...] = h
    hp_ref[...] = h * dinv[:, None]
    dinv_ref[...] = dinv
    ia_ref[...] = 1.0 / jnp.maximum(ca_ref[...], 1.0)
    ib_ref[...] = 1.0 / jnp.maximum(cb_ref[...], 1.0)


def _tc_prolog(x, w, b, dga, ca, cb):
    blk = 256
    grid = pl.cdiv(N, blk)
    vspec = pl.BlockSpec((blk,), lambda i: (i,))
    return pl.pallas_call(
        _prolog_body,
        grid=(grid,),
        in_specs=[
            pl.BlockSpec((blk, D), lambda i: (i, 0)),
            pl.BlockSpec((D, D), lambda i: (0, 0)),
            pl.BlockSpec((D,), lambda i: (0,)),
            vspec, vspec, vspec,
        ],
        out_specs=[
            pl.BlockSpec((blk, D), lambda i: (i, 0)),
            pl.BlockSpec((blk, D), lambda i: (i, 0)),
            vspec, vspec, vspec,
        ],
        out_shape=[
            jax.ShapeDtypeStruct((N, D), _f32),
            jax.ShapeDtypeStruct((N, D), _f32),
            jax.ShapeDtypeStruct((N,), _f32),
            jax.ShapeDtypeStruct((N,), _f32),
            jax.ShapeDtypeStruct((N,), _f32),
        ],
    )(x, w, b, dga, ca, cb)


def _layer_body(ma_ref, md_ref, mc_ref, h_ref, hp_ref, dinv_ref, ia_ref,
                ib_ref, u_ref, bias_ref, g_ref, lb_ref, hn_ref, hpn_ref):
    dinv = dinv_ref[...][:, None]
    a = (ma_ref[...] + hp_ref[...]) * dinv
    b = md_ref[...] * ia_ref[...][:, None]
    cc = mc_ref[...] * ib_ref[...][:, None]
    x = jnp.concatenate([a, b, cc, h_ref[...]], axis=1)
    out = jnp.dot(x, u_ref[...], preferred_element_type=_f32) + bias_ref[...]
    out = jnp.maximum(out, 0.0)
    mu = jnp.mean(out, axis=1, keepdims=True)
    var = jnp.mean((out - mu) ** 2, axis=1, keepdims=True)
    hn = (out - mu) * lax.rsqrt(var + 1e-5) * g_ref[...] + lb_ref[...]
    hn_ref[...] = hn
    hpn_ref[...] = hn * dinv


def _tc_layer(ma, md, mc, h, hp, dinv, ia, ib, u, bias, g, lb):
    blk = 256
    grid = pl.cdiv(N, blk)
    mspec = pl.BlockSpec((blk, D), lambda i: (i, 0))
    vspec = pl.BlockSpec((blk,), lambda i: (i,))
    return pl.pallas_call(
        _layer_body,
        grid=(grid,),
        in_specs=[
            mspec, mspec, mspec, mspec, mspec,
            vspec, vspec, vspec,
            pl.BlockSpec((4 * D, D), lambda i: (0, 0)),
            pl.BlockSpec((D,), lambda i: (0,)),
            pl.BlockSpec((D,), lambda i: (0,)),
            pl.BlockSpec((D,), lambda i: (0,)),
        ],
        out_specs=[mspec, mspec],
        out_shape=[
            jax.ShapeDtypeStruct((N, D), _f32),
            jax.ShapeDtypeStruct((N, D), _f32),
        ],
    )(ma, md, mc, h, hp, dinv, ia, ib, u, bias, g, lb)


def _epi_body(h_ref, bn_ref, attw_ref, outw_ref, outb_ref, o_ref):
    h = h_ref[...]
    logits = jnp.dot(h, attw_ref[...], preferred_element_type=_f32)[:, 0]
    p = jnp.exp(logits - jnp.max(logits))
    p = p / jnp.sum(p)
    wtd = h * p[:, None]
    oh = (bn_ref[...][:, None] ==
          lax.broadcasted_iota(_i32, (N, G), 1)).astype(_f32)
    gr = lax.dot_general(oh, wtd, (((0,), (0,)), ((), ())),
                         preferred_element_type=_f32)
    gc = lax.dot_general(oh, p[:, None], (((0,), (0,)), ((), ())),
                         preferred_element_type=_f32)
    gr = gr / jnp.maximum(gc, 1e-6)
    o_ref[...] = jnp.dot(gr, outw_ref[...],
                         preferred_element_type=_f32) + outb_ref[...]


def _tc_epilogue(h, bn, attw, outw, outb):
    return pl.pallas_call(
        _epi_body,
        out_shape=jax.ShapeDtypeStruct((G, D), _f32),
    )(h, bn, attw, outw, outb)


def kernel(x_node, edge_index_ast, edge_index_df, edge_index_cf, batch_node,
           proj_W, proj_b, gcn_W, gcn_b, sgA_Wl, sgA_bl, sgA_Wr,
           sgB_Wl, sgB_bl, sgB_Wr, ln_g, ln_b, att_W, att_b, out_W, out_b):
    (lst0, lst1, lst2, cnt0, cnt1, cnt2,
     deg0, deg1, deg2) = _sc_bucketize(
         edge_index_ast[0], edge_index_ast[1],
         edge_index_df[0], edge_index_df[1],
         edge_index_cf[0], edge_index_cf[1])
    dga = deg0[:N, 0]
    ca = deg1[:N, 0]
    cb = deg2[:N, 0]
    import os as _os
    if _os.environ.get("_BISECT") == "A":
        return (dga[:G, None] + ca[:G, None] + cb[:G, None]
                + jnp.zeros((G, D), _f32)
                + lst0[0, 0] + cnt0[0, 0])

    h, hp, dinv, ia, ib = _tc_prolog(x_node, proj_W, proj_b, dga, ca, cb)

    for l in range(gcn_W.shape[0]):
        u = jnp.concatenate(
            [gcn_W[l], sgA_Wl[l], sgB_Wl[l], sgA_Wr[l] + sgB_Wr[l]], axis=0)
        bias = gcn_b[l] + sgA_bl[l] + sgB_bl[l]
        ma, md, mc = _sc_segsum3(h, hp, lst0, lst1, lst2, cnt0, cnt1, cnt2)
        ma = ma.reshape(NR, D)[:N]
        md = md.reshape(NR, D)[:N]
        mc = mc.reshape(NR, D)[:N]
        h, hp = _tc_layer(ma, md, mc, h, hp, dinv, ia, ib,
                          u, bias, ln_g, ln_b)

    return _tc_epilogue(h, batch_node, att_W, out_W, out_b)


# speedup vs baseline: 1.2674x; 1.0036x over previous
import functools

import jax
import jax.numpy as jnp
from jax import lax
from jax.experimental import pallas as pl
from jax.experimental.pallas import tpu as pltpu
from jax.experimental.pallas import tpu_sc as plsc

N = 10000
D = 256
G = 64
E = 160000
NB = 32
R = 320
NR = NB * R
RT = R + 8
K = 128
C = 2000
EPAD = E + K

_mesh = plsc.VectorSubcoreMesh(core_axis_name="c", subcore_axis_name="s",
                               num_cores=2, num_subcores=16)

_i32 = jnp.int32
_f32 = jnp.float32


@functools.partial(
    pl.kernel,
    out_type=(
        jax.ShapeDtypeStruct((NB, EPAD), _i32),
        jax.ShapeDtypeStruct((NB, EPAD), _i32),
        jax.ShapeDtypeStruct((NB, EPAD), _i32),
        jax.ShapeDtypeStruct((NB, 16), _i32),
        jax.ShapeDtypeStruct((NB, 16), _i32),
        jax.ShapeDtypeStruct((NB, 16), _i32),
        jax.ShapeDtypeStruct((NR, 16), _f32),
        jax.ShapeDtypeStruct((NR, 16), _f32),
        jax.ShapeDtypeStruct((NR, 16), _f32),
    ),
    mesh=_mesh,
    scratch_types=[
        pltpu.VMEM((C,), _i32),
        pltpu.VMEM((C,), _i32),
        pltpu.VMEM((C,), _i32),
        pltpu.VMEM((C,), _i32),
        pltpu.VMEM((C + 304,), _i32),
        pltpu.VMEM((16,), _i32),
        pltpu.VMEM((K + 16,), _i32),
        pltpu.VMEM((RT, 16), _f32),
        pltpu.SMEM((K,), _i32),
        pltpu.SemaphoreType.DMA,
        pltpu.SemaphoreType.DMA,
        pltpu.SemaphoreType.DMA,
        pltpu.SemaphoreType.DMA,
    ],
    compiler_params=pltpu.CompilerParams(needs_layout_passes=False, disable_bounds_checks=True),
)
def _sc_bucketize(es0, ed0, es1, ed1, es2, ed2, lst0, lst1, lst2,
                  cnt0, cnt1, cnt2, deg0, deg1, deg2, srcb0, dstb0,
                  srcb1, dstb1, listb, cvec, kbuf, acc16, smk,
                  sema0, semb0, sema1, semb1):
    c = lax.axis_index("c")
    s = lax.axis_index("s")
    w = c * 16 + s
    lo = w * R

    for es, ed, lst, cnt, deg in ((es0, ed0, lst0, cnt0, deg0),
                                  (es1, ed1, lst1, cnt1, deg1),
                                  (es2, ed2, lst2, cnt2, deg2)):
        def mk_scan(srcb, dstb):
            def vec_body(j, posv):
                sv = srcb[pl.ds(j * 16, 16)]
                dv = dstb[pl.ds(j * 16, 16)]
                dl = dv - lo
                m = (dl >= 0) & (dl < R)
                pk = (sv << 9) | jnp.where(m, dl, 0)
                kin = m.astype(_i32)
                excl = plsc.cumsum(kin) - kin
                idx = jnp.where(m, posv + excl, C + 303)
                plsc.store_scatter(listb, [idx], pk)
                return posv + plsc.all_reduce_population_count(m)
            return vec_body

        def flush_fn(pos, outb):
            nblk = pos // K

            def flush(b, _):
                pltpu.sync_copy(listb.at[pl.ds(b * K, K)],
                                lst.at[w, pl.ds((outb + b) * K, K)])
                return 0
            lax.fori_loop(0, nblk, flush, 0)
            rs = nblk * K

            def shift(v, _):
                tmp = listb[pl.ds(rs + v * 16, 16)]
                listb[pl.ds(v * 16, 16)] = tmp
                return 0
            lax.fori_loop(0, K // 16, shift, 0)
            return pos - rs, outb + nblk

        def pair_body(t, carry):
            pos, outb = carry
            e0 = t * 2 * C
            d0a = pltpu.async_copy(es.at[pl.ds(e0, C)], srcb0, sema0)
            d0b = pltpu.async_copy(ed.at[pl.ds(e0, C)], dstb0, semb0)
            d1a = pltpu.async_copy(es.at[pl.ds(e0 + C, C)], srcb1, sema1)
            d1b = pltpu.async_copy(ed.at[pl.ds(e0 + C, C)], dstb1, semb1)
            d0a.wait()
            d0b.wait()
            posv = lax.fori_loop(0, C // 16, mk_scan(srcb0, dstb0),
                                 jnp.zeros((16,), _i32) + pos, unroll=4)
            pos, outb = flush_fn(posv[0], outb)
            d1a.wait()
            d1b.wait()
            posv = lax.fori_loop(0, C // 16, mk_scan(srcb1, dstb1),
                                 jnp.zeros((16,), _i32) + pos, unroll=4)
            pos, outb = flush_fn(posv[0], outb)
            return pos, outb

        pos, outb = lax.fori_loop(0, E // C // 2, pair_body,
                                  (jnp.asarray(0, _i32), jnp.asarray(0, _i32)))

        sent = jnp.full((16,), R, _i32)

        def pad(v, _):
            listb[pl.ds(pos + v * 16, 16)] = sent
            return 0
        lax.fori_loop(0, K // 16, pad, 0)
        nblk2 = (pos + K - 1) // K

        def flush2(b, _):
            pltpu.sync_copy(listb.at[pl.ds(b * K, K)],
                            lst.at[w, pl.ds((outb + b) * K, K)])
            return 0
        lax.fori_loop(0, nblk2, flush2, 0)
        total = (outb + nblk2) * K
        cvec[...] = jnp.zeros((16,), _i32) + total
        pltpu.sync_copy(cvec, cnt.at[w])

        def za16(i, _):
            acc16[i, pl.ds(0, 16)] = jnp.zeros((16,), _f32)
            return 0
        lax.fori_loop(0, RT, za16, 0)

        def count_chunk(t, _):
            pltpu.sync_copy(lst.at[w, pl.ds(t * K, K)], kbuf.at[pl.ds(0, K)])

            def ext(k, _):
                smk[k] = kbuf[pl.ds(k, 16)][0]
                return 0
            lax.fori_loop(0, K, ext, 0, unroll=8)

            def one(k, _):
                plsc.addupdate(acc16.at[smk[k] & 511, pl.ds(0, 16)],
                               jnp.ones((16,), _f32))
                return 0
            lax.fori_loop(0, K, one, 0, unroll=4)
            return 0
        lax.fori_loop(0, total // K, count_chunk, 0)
        pltpu.sync_copy(acc16.at[pl.ds(0, R)], deg.at[pl.ds(w * R, R)])


KB = 64


@functools.partial(
    pl.kernel,
    out_type=(
        jax.ShapeDtypeStruct((NR * D,), _f32),
        jax.ShapeDtypeStruct((NR * D,), _f32),
        jax.ShapeDtypeStruct((NR * D,), _f32),
    ),
    mesh=_mesh,
    scratch_types=[
        pltpu.VMEM((KB + 16,), _i32),
        pltpu.VMEM((KB + 16,), _i32),
        pltpu.VMEM((KB,), _i32),
        pltpu.VMEM((KB,), _i32),
        pltpu.VMEM((KB, D), _f32),
        pltpu.VMEM((KB, D), _f32),
        pltpu.VMEM((16,), _i32),
        pltpu.VMEM((RT * D,), _f32),
        pltpu.SMEM((KB,), _i32),
        pltpu.SMEM((KB,), _i32),
        pltpu.SemaphoreType.DMA,
        pltpu.SemaphoreType.DMA,
    ],
    compiler_params=pltpu.CompilerParams(needs_layout_passes=False, disable_bounds_checks=True),
)
def _sc_segsum3(h, hp, lst0, lst1, lst2, cnt0, cnt1, cnt2,
                o0, o1, o2, pkb0, pkb1, gsrc0, gsrc1, rows0, rows1,
                cntv, acc, sm0, sm1, sem0, sem1):
    c = lax.axis_index("c")
    s = lax.axis_index("s")
    w = c * 16 + s

    for tab, lst, cnt, out in ((hp, lst0, cnt0, o0),
                               (h, lst1, cnt1, o1),
                               (h, lst2, cnt2, o2)):
        def za(i, _):
            acc[pl.ds(i * 16, 16)] = jnp.zeros((16,), _f32)
            return 0
        lax.fori_loop(0, RT * D // 16, za, 0, unroll=4)

        pltpu.sync_copy(cnt.at[w], cntv)
        pairs = cntv[...][0] // (2 * KB)

        def mk_acc(sm, rows):
            def grp(j, _):
                for i in range(8):
                    k = j * 8 + i
                    b = (sm[k] & 511) << 8
                    for cc in range(16):
                        plsc.addupdate(acc.at[pl.ds(b + cc * 16, 16)],
                                       rows[k, pl.ds(cc * 16, 16)])
                return 0
            return grp

        def mk_ext(pkb, sm):
            def ext(k, _):
                sm[k] = pkb[pl.ds(k, 16)][0]
                return 0
            return ext

        def pair_chunk(t, _):
            e0 = t * 2 * KB
            pltpu.sync_copy(lst.at[w, pl.ds(e0, KB)], pkb0.at[pl.ds(0, KB)])

            def up0(j, _):
                gsrc0[pl.ds(j * 16, 16)] = pkb0[pl.ds(j * 16, 16)] >> 9
                return 0
            lax.fori_loop(0, KB // 16, up0, 0, unroll=4)
            d0 = pltpu.async_copy(tab.at[gsrc0], rows0, sem0)
            lax.fori_loop(0, KB, mk_ext(pkb0, sm0), 0, unroll=8)

            pltpu.sync_copy(lst.at[w, pl.ds(e0 + KB, KB)],
                            pkb1.at[pl.ds(0, KB)])

            def up1(j, _):
                gsrc1[pl.ds(j * 16, 16)] = pkb1[pl.ds(j * 16, 16)] >> 9
                return 0
            lax.fori_loop(0, KB // 16, up1, 0, unroll=4)
            d1 = pltpu.async_copy(tab.at[gsrc1], rows1, sem1)
            lax.fori_loop(0, KB, mk_ext(pkb1, sm1), 0, unroll=8)

            d0.wait()
            lax.fori_loop(0, KB // 8, mk_acc(sm0, rows0), 0)
            d1.wait()
            lax.fori_loop(0, KB // 8, mk_acc(sm1, rows1), 0)
            return 0
        lax.fori_loop(0, pairs, pair_chunk, 0)

        pltpu.sync_copy(acc.at[pl.ds(0, R * D)],
                        out.at[pl.ds(w * R * D, R * D)])


def _prolog_body(x_ref, w_ref, b_ref, dga_ref, ca_ref, cb_ref,
                 h_ref, hp_ref, dinv_ref, ia_ref, ib_ref):
    h = jnp.maximum(jnp.dot(x_ref[...], w_ref[...],
                            preferred_element_type=_f32) + b_ref[...], 0.0)
    dinv = lax.rsqrt(dga_ref[...] + 1.0)
    h_ref[...] = h
    hp_ref[...] = h * dinv[:, None]
    dinv_ref[...] = dinv
    ia_ref[...] = 1.0 / jnp.maximum(ca_ref[...], 1.0)
    ib_ref[...] = 1.0 / jnp.maximum(cb_ref[...], 1.0)


def _tc_prolog(x, w, b, dga, ca, cb):
    blk = 256
    grid = pl.cdiv(N, blk)
    vspec = pl.BlockSpec((blk,), lambda i: (i,))
    return pl.pallas_call(
        _prolog_body,
        grid=(grid,),
        in_specs=[
            pl.BlockSpec((blk, D), lambda i: (i, 0)),
            pl.BlockSpec((D, D), lambda i: (0, 0)),
            pl.BlockSpec((D,), lambda i: (0,)),
            vspec, vspec, vspec,
        ],
        out_specs=[
            pl.BlockSpec((blk, D), lambda i: (i, 0)),
            pl.BlockSpec((blk, D), lambda i: (i, 0)),
            vspec, vspec, vspec,
        ],
        out_shape=[
            jax.ShapeDtypeStruct((N, D), _f32),
            jax.ShapeDtypeStruct((N, D), _f32),
            jax.ShapeDtypeStruct((N,), _f32),
            jax.ShapeDtypeStruct((N,), _f32),
            jax.ShapeDtypeStruct((N,), _f32),
        ],
    )(x, w, b, dga, ca, cb)


def _layer_body(ma_ref, md_ref, mc_ref, h_ref, hp_ref, dinv_ref, ia_ref,
                ib_ref, u_ref, bias_ref, g_ref, lb_ref, hn_ref, hpn_ref):
    dinv = dinv_ref[...][:, None]
    a = (ma_ref[...] + hp_ref[...]) * dinv
    b = md_ref[...] * ia_ref[...][:, None]
    cc = mc_ref[...] * ib_ref[...][:, None]
    x = jnp.concatenate([a, b, cc, h_ref[...]], axis=1)
    out = jnp.dot(x, u_ref[...], preferred_element_type=_f32) + bias_ref[...]
    out = jnp.maximum(out, 0.0)
    mu = jnp.mean(out, axis=1, keepdims=True)
    var = jnp.mean((out - mu) ** 2, axis=1, keepdims=True)
    hn = (out - mu) * lax.rsqrt(var + 1e-5) * g_ref[...] + lb_ref[...]
    hn_ref[...] = hn
    hpn_ref[...] = hn * dinv


def _tc_layer(ma, md, mc, h, hp, dinv, ia, ib, u, bias, g, lb):
    blk = 256
    grid = pl.cdiv(N, blk)
    mspec = pl.BlockSpec((blk, D), lambda i: (i, 0))
    vspec = pl.BlockSpec((blk,), lambda i: (i,))
    return pl.pallas_call(
        _layer_body,
        grid=(grid,),
        in_specs=[
            mspec, mspec, mspec, mspec, mspec,
            vspec, vspec, vspec,
            pl.BlockSpec((4 * D, D), lambda i: (0, 0)),
            pl.BlockSpec((D,), lambda i: (0,)),
            pl.BlockSpec((D,), lambda i: (0,)),
            pl.BlockSpec((D,), lambda i: (0,)),
        ],
        out_specs=[mspec, mspec],
        out_shape=[
            jax.ShapeDtypeStruct((N, D), _f32),
            jax.ShapeDtypeStruct((N, D), _f32),
        ],
    )(ma, md, mc, h, hp, dinv, ia, ib, u, bias, g, lb)


def _epi_body(h_ref, bn_ref, attw_ref, outw_ref, outb_ref, o_ref):
    h = h_ref[...]
    logits = jnp.dot(h, attw_ref[...], preferred_element_type=_f32)[:, 0]
    p = jnp.exp(logits - jnp.max(logits))
    p = p / jnp.sum(p)
    wtd = h * p[:, None]
    oh = (bn_ref[...][:, None] ==
          lax.broadcasted_iota(_i32, (N, G), 1)).astype(_f32)
    gr = lax.dot_general(oh, wtd, (((0,), (0,)), ((), ())),
                         preferred_element_type=_f32)
    gc = lax.dot_general(oh, p[:, None], (((0,), (0,)), ((), ())),
                         preferred_element_type=_f32)
    gr = gr / jnp.maximum(gc, 1e-6)
    o_ref[...] = jnp.dot(gr, outw_ref[...],
                         preferred_element_type=_f32) + outb_ref[...]


def _tc_epilogue(h, bn, attw, outw, outb):
    return pl.pallas_call(
        _epi_body,
        out_shape=jax.ShapeDtypeStruct((G, D), _f32),
    )(h, bn, attw, outw, outb)


def kernel(x_node, edge_index_ast, edge_index_df, edge_index_cf, batch_node,
           proj_W, proj_b, gcn_W, gcn_b, sgA_Wl, sgA_bl, sgA_Wr,
           sgB_Wl, sgB_bl, sgB_Wr, ln_g, ln_b, att_W, att_b, out_W, out_b):
    (lst0, lst1, lst2, cnt0, cnt1, cnt2,
     deg0, deg1, deg2) = _sc_bucketize(
         edge_index_ast[0], edge_index_ast[1],
         edge_index_df[0], edge_index_df[1],
         edge_index_cf[0], edge_index_cf[1])
    dga = deg0[:N, 0]
    ca = deg1[:N, 0]
    cb = deg2[:N, 0]
    import os as _os
    if _os.environ.get("_BISECT") == "A":
        return (dga[:G, None] + ca[:G, None] + cb[:G, None]
                + jnp.zeros((G, D), _f32)
                + lst0[0, 0] + cnt0[0, 0])

    h, hp, dinv, ia, ib = _tc_prolog(x_node, proj_W, proj_b, dga, ca, cb)

    for l in range(gcn_W.shape[0]):
        u = jnp.concatenate(
            [gcn_W[l], sgA_Wl[l], sgB_Wl[l], sgA_Wr[l] + sgB_Wr[l]], axis=0)
        bias = gcn_b[l] + sgA_bl[l] + sgB_bl[l]
        ma, md, mc = _sc_segsum3(h, hp, lst0, lst1, lst2, cnt0, cnt1, cnt2)
        ma = ma.reshape(NR, D)[:N]
        md = md.reshape(NR, D)[:N]
        mc = mc.reshape(NR, D)[:N]
        h, hp = _tc_layer(ma, md, mc, h, hp, dinv, ia, ib,
                          u, bias, ln_g, ln_b)

    return _tc_epilogue(h, batch_node, att_W, out_W, out_b)


# speedup vs baseline: 1.3880x; 1.0951x over previous
import functools

import jax
import jax.numpy as jnp
from jax import lax
from jax.experimental import pallas as pl
from jax.experimental.pallas import tpu as pltpu
from jax.experimental.pallas import tpu_sc as plsc

N = 10000
D = 256
G = 64
E = 160000
NB = 32
R = 320
NR = NB * R
RT = R + 8
K = 128
C = 2000
EPAD = E + K

_mesh = plsc.VectorSubcoreMesh(core_axis_name="c", subcore_axis_name="s",
                               num_cores=2, num_subcores=16)

_i32 = jnp.int32
_f32 = jnp.float32


@functools.partial(
    pl.kernel,
    out_type=(
        jax.ShapeDtypeStruct((NB, EPAD), _i32),
        jax.ShapeDtypeStruct((NB, EPAD), _i32),
        jax.ShapeDtypeStruct((NB, EPAD), _i32),
        jax.ShapeDtypeStruct((NB, 16), _i32),
        jax.ShapeDtypeStruct((NB, 16), _i32),
        jax.ShapeDtypeStruct((NB, 16), _i32),
        jax.ShapeDtypeStruct((NR, 16), _f32),
        jax.ShapeDtypeStruct((NR, 16), _f32),
        jax.ShapeDtypeStruct((NR, 16), _f32),
    ),
    mesh=_mesh,
    scratch_types=[
        pltpu.VMEM((C,), _i32),
        pltpu.VMEM((C,), _i32),
        pltpu.VMEM((C,), _i32),
        pltpu.VMEM((C,), _i32),
        pltpu.VMEM((C + 304,), _i32),
        pltpu.VMEM((16,), _i32),
        pltpu.VMEM((K + 16,), _i32),
        pltpu.VMEM((RT, 16), _f32),
        pltpu.SMEM((K,), _i32),
        pltpu.SemaphoreType.DMA,
        pltpu.SemaphoreType.DMA,
        pltpu.SemaphoreType.DMA,
        pltpu.SemaphoreType.DMA,
    ],
    compiler_params=pltpu.CompilerParams(needs_layout_passes=False, disable_bounds_checks=True),
)
def _sc_bucketize(es0, ed0, es1, ed1, es2, ed2, lst0, lst1, lst2,
                  cnt0, cnt1, cnt2, deg0, deg1, deg2, srcb0, dstb0,
                  srcb1, dstb1, listb, cvec, kbuf, acc16, smk,
                  sema0, semb0, sema1, semb1):
    c = lax.axis_index("c")
    s = lax.axis_index("s")
    w = c * 16 + s
    lo = w * R

    for es, ed, lst, cnt, deg in ((es0, ed0, lst0, cnt0, deg0),
                                  (es1, ed1, lst1, cnt1, deg1),
                                  (es2, ed2, lst2, cnt2, deg2)):
        def mk_scan(srcb, dstb):
            def vec_body(j, posv):
                sv = srcb[pl.ds(j * 16, 16)]
                dv = dstb[pl.ds(j * 16, 16)]
                dl = dv - lo
                m = (dl >= 0) & (dl < R)
                pk = (sv << 9) | jnp.where(m, dl, 0)
                kin = m.astype(_i32)
                excl = plsc.cumsum(kin) - kin
                idx = jnp.where(m, posv + excl, C + 303)
                plsc.store_scatter(listb, [idx], pk)
                return posv + plsc.all_reduce_population_count(m)
            return vec_body

        def flush_fn(pos, outb):
            nblk = pos // K

            def flush(b, _):
                pltpu.sync_copy(listb.at[pl.ds(b * K, K)],
                                lst.at[w, pl.ds((outb + b) * K, K)])
                return 0
            lax.fori_loop(0, nblk, flush, 0)
            rs = nblk * K

            def shift(v, _):
                tmp = listb[pl.ds(rs + v * 16, 16)]
                listb[pl.ds(v * 16, 16)] = tmp
                return 0
            lax.fori_loop(0, K // 16, shift, 0)
            return pos - rs, outb + nblk

        def pair_body(t, carry):
            pos, outb = carry
            e0 = t * 2 * C
            d0a = pltpu.async_copy(es.at[pl.ds(e0, C)], srcb0, sema0)
            d0b = pltpu.async_copy(ed.at[pl.ds(e0, C)], dstb0, semb0)
            d1a = pltpu.async_copy(es.at[pl.ds(e0 + C, C)], srcb1, sema1)
            d1b = pltpu.async_copy(ed.at[pl.ds(e0 + C, C)], dstb1, semb1)
            d0a.wait()
            d0b.wait()
            posv = lax.fori_loop(0, C // 16, mk_scan(srcb0, dstb0),
                                 jnp.zeros((16,), _i32) + pos, unroll=4)
            pos, outb = flush_fn(posv[0], outb)
            d1a.wait()
            d1b.wait()
            posv = lax.fori_loop(0, C // 16, mk_scan(srcb1, dstb1),
                                 jnp.zeros((16,), _i32) + pos, unroll=4)
            pos, outb = flush_fn(posv[0], outb)
            return pos, outb

        pos, outb = lax.fori_loop(0, E // C // 2, pair_body,
                                  (jnp.asarray(0, _i32), jnp.asarray(0, _i32)))

        sent = jnp.full((16,), R, _i32)

        def pad(v, _):
            listb[pl.ds(pos + v * 16, 16)] = sent
            return 0
        lax.fori_loop(0, K // 16, pad, 0)
        nblk2 = (pos + K - 1) // K

        def flush2(b, _):
            pltpu.sync_copy(listb.at[pl.ds(b * K, K)],
                            lst.at[w, pl.ds((outb + b) * K, K)])
            return 0
        lax.fori_loop(0, nblk2, flush2, 0)
        total = (outb + nblk2) * K
        cvec[...] = jnp.zeros((16,), _i32) + total
        pltpu.sync_copy(cvec, cnt.at[w])

        def za16(i, _):
            acc16[i, pl.ds(0, 16)] = jnp.zeros((16,), _f32)
            return 0
        lax.fori_loop(0, RT, za16, 0)

        def count_chunk(t, _):
            pltpu.sync_copy(lst.at[w, pl.ds(t * K, K)], kbuf.at[pl.ds(0, K)])

            def ext(k, _):
                smk[k] = kbuf[pl.ds(k, 16)][0]
                return 0
            lax.fori_loop(0, K, ext, 0, unroll=8)

            def one(k, _):
                plsc.addupdate(acc16.at[smk[k] & 511, pl.ds(0, 16)],
                               jnp.ones((16,), _f32))
                return 0
            lax.fori_loop(0, K, one, 0, unroll=4)
            return 0
        lax.fori_loop(0, total // K, count_chunk, 0)
        pltpu.sync_copy(acc16.at[pl.ds(0, R)], deg.at[pl.ds(w * R, R)])


KB = 64


@functools.partial(
    pl.kernel,
    out_type=(
        jax.ShapeDtypeStruct((NR * D,), _f32),
        jax.ShapeDtypeStruct((NR * D,), _f32),
        jax.ShapeDtypeStruct((NR * D,), _f32),
    ),
    mesh=_mesh,
    scratch_types=[
        pltpu.VMEM((KB + 16,), _i32),
        pltpu.VMEM((KB + 16,), _i32),
        pltpu.VMEM((KB,), _i32),
        pltpu.VMEM((KB,), _i32),
        pltpu.VMEM((KB, D // 2), _i32),
        pltpu.VMEM((KB, D // 2), _i32),
        pltpu.VMEM((16,), _i32),
        pltpu.VMEM((RT * D,), _f32),
        pltpu.SMEM((KB,), _i32),
        pltpu.SMEM((KB,), _i32),
        pltpu.SemaphoreType.DMA,
        pltpu.SemaphoreType.DMA,
    ],
    compiler_params=pltpu.CompilerParams(needs_layout_passes=False, disable_bounds_checks=True),
)
def _sc_segsum3(h, hp, lst0, lst1, lst2, cnt0, cnt1, cnt2,
                o0, o1, o2, pkb0, pkb1, gsrc0, gsrc1, rows0, rows1,
                cntv, acc, sm0, sm1, sem0, sem1):
    c = lax.axis_index("c")
    s = lax.axis_index("s")
    w = c * 16 + s

    for tab, lst, cnt, out in ((hp, lst0, cnt0, o0),
                               (h, lst1, cnt1, o1),
                               (h, lst2, cnt2, o2)):
        def za(i, _):
            acc[pl.ds(i * 16, 16)] = jnp.zeros((16,), _f32)
            return 0
        lax.fori_loop(0, RT * D // 16, za, 0, unroll=4)

        pltpu.sync_copy(cnt.at[w], cntv)
        pairs = cntv[...][0] // (2 * KB)

        def mk_acc(sm, rows):
            def grp(j, _):
                for i in range(8):
                    k = j * 8 + i
                    b = (sm[k] & 511) << 8
                    for cc in range(8):
                        v = plsc.bitcast(rows[k, pl.ds(cc * 16, 16)],
                                         jnp.bfloat16)
                        ev, od = plsc.unpack(
                            v, format=plsc.PackFormat.INTERLEAVED)
                        plsc.addupdate(acc.at[pl.ds(b + cc * 32, 16)], ev)
                        plsc.addupdate(acc.at[pl.ds(b + cc * 32 + 16, 16)], od)
                return 0
            return grp

        def mk_ext(pkb, sm):
            def ext(k, _):
                sm[k] = pkb[pl.ds(k, 16)][0]
                return 0
            return ext

        def pair_chunk(t, _):
            e0 = t * 2 * KB
            pltpu.sync_copy(lst.at[w, pl.ds(e0, KB)], pkb0.at[pl.ds(0, KB)])

            def up0(j, _):
                gsrc0[pl.ds(j * 16, 16)] = pkb0[pl.ds(j * 16, 16)] >> 9
                return 0
            lax.fori_loop(0, KB // 16, up0, 0, unroll=4)
            d0 = pltpu.async_copy(tab.at[gsrc0], rows0, sem0)
            lax.fori_loop(0, KB, mk_ext(pkb0, sm0), 0, unroll=8)

            pltpu.sync_copy(lst.at[w, pl.ds(e0 + KB, KB)],
                            pkb1.at[pl.ds(0, KB)])

            def up1(j, _):
                gsrc1[pl.ds(j * 16, 16)] = pkb1[pl.ds(j * 16, 16)] >> 9
                return 0
            lax.fori_loop(0, KB // 16, up1, 0, unroll=4)
            d1 = pltpu.async_copy(tab.at[gsrc1], rows1, sem1)
            lax.fori_loop(0, KB, mk_ext(pkb1, sm1), 0, unroll=8)

            d0.wait()
            lax.fori_loop(0, KB // 8, mk_acc(sm0, rows0), 0)
            d1.wait()
            lax.fori_loop(0, KB // 8, mk_acc(sm1, rows1), 0)
            return 0
        lax.fori_loop(0, pairs, pair_chunk, 0)

        pltpu.sync_copy(acc.at[pl.ds(0, R * D)],
                        out.at[pl.ds(w * R * D, R * D)])


def _prolog_body(x_ref, w_ref, b_ref, dga_ref, ca_ref, cb_ref,
                 h_ref, hbf_ref, hpbf_ref, dinv_ref, ia_ref, ib_ref):
    h = jnp.maximum(jnp.dot(x_ref[...], w_ref[...],
                            preferred_element_type=_f32) + b_ref[...], 0.0)
    dinv = lax.rsqrt(dga_ref[...] + 1.0)
    h_ref[...] = h
    hbf_ref[...] = h.astype(jnp.bfloat16)
    hpbf_ref[...] = (h * dinv[:, None]).astype(jnp.bfloat16)
    dinv_ref[...] = dinv
    ia_ref[...] = 1.0 / jnp.maximum(ca_ref[...], 1.0)
    ib_ref[...] = 1.0 / jnp.maximum(cb_ref[...], 1.0)


def _tc_prolog(x, w, b, dga, ca, cb):
    blk = 256
    grid = pl.cdiv(N, blk)
    vspec = pl.BlockSpec((blk,), lambda i: (i,))
    mspec = pl.BlockSpec((blk, D), lambda i: (i, 0))
    return pl.pallas_call(
        _prolog_body,
        grid=(grid,),
        in_specs=[
            mspec,
            pl.BlockSpec((D, D), lambda i: (0, 0)),
            pl.BlockSpec((D,), lambda i: (0,)),
            vspec, vspec, vspec,
        ],
        out_specs=[mspec, mspec, mspec, vspec, vspec, vspec],
        out_shape=[
            jax.ShapeDtypeStruct((N, D), _f32),
            jax.ShapeDtypeStruct((N, D), jnp.bfloat16),
            jax.ShapeDtypeStruct((N, D), jnp.bfloat16),
            jax.ShapeDtypeStruct((N,), _f32),
            jax.ShapeDtypeStruct((N,), _f32),
            jax.ShapeDtypeStruct((N,), _f32),
        ],
    )(x, w, b, dga, ca, cb)


def _layer_body(ma_ref, md_ref, mc_ref, h_ref, dinv_ref, ia_ref,
                ib_ref, u_ref, bias_ref, g_ref, lb_ref,
                hn_ref, hnbf_ref, hpnbf_ref):
    dinv = dinv_ref[...][:, None]
    h = h_ref[...]
    a = ma_ref[...] * dinv
    b = md_ref[...] * ia_ref[...][:, None]
    cc = mc_ref[...] * ib_ref[...][:, None]
    x = jnp.concatenate([a, b, cc, h * (dinv * dinv), h], axis=1)
    out = jnp.dot(x, u_ref[...], preferred_element_type=_f32) + bias_ref[...]
    out = jnp.maximum(out, 0.0)
    mu = jnp.mean(out, axis=1, keepdims=True)
    var = jnp.mean((out - mu) ** 2, axis=1, keepdims=True)
    hn = (out - mu) * lax.rsqrt(var + 1e-5) * g_ref[...] + lb_ref[...]
    hn_ref[...] = hn
    hnbf_ref[...] = hn.astype(jnp.bfloat16)
    hpnbf_ref[...] = (hn * dinv).astype(jnp.bfloat16)


def _tc_layer(ma, md, mc, h, dinv, ia, ib, u, bias, g, lb):
    blk = 256
    grid = pl.cdiv(N, blk)
    mspec = pl.BlockSpec((blk, D), lambda i: (i, 0))
    vspec = pl.BlockSpec((blk,), lambda i: (i,))
    return pl.pallas_call(
        _layer_body,
        grid=(grid,),
        in_specs=[
            mspec, mspec, mspec, mspec,
            vspec, vspec, vspec,
            pl.BlockSpec((5 * D, D), lambda i: (0, 0)),
            pl.BlockSpec((D,), lambda i: (0,)),
            pl.BlockSpec((D,), lambda i: (0,)),
            pl.BlockSpec((D,), lambda i: (0,)),
        ],
        out_specs=[mspec, mspec, mspec],
        out_shape=[
            jax.ShapeDtypeStruct((N, D), _f32),
            jax.ShapeDtypeStruct((N, D), jnp.bfloat16),
            jax.ShapeDtypeStruct((N, D), jnp.bfloat16),
        ],
    )(ma, md, mc, h, dinv, ia, ib, u, bias, g, lb)


def _epi_body(h_ref, bn_ref, attw_ref, outw_ref, outb_ref, o_ref):
    h = h_ref[...]
    logits = jnp.dot(h, attw_ref[...], preferred_element_type=_f32)[:, 0]
    p = jnp.exp(logits - jnp.max(logits))
    p = p / jnp.sum(p)
    wtd = h * p[:, None]
    oh = (bn_ref[...][:, None] ==
          lax.broadcasted_iota(_i32, (N, G), 1)).astype(_f32)
    gr = lax.dot_general(oh, wtd, (((0,), (0,)), ((), ())),
                         preferred_element_type=_f32)
    gc = lax.dot_general(oh, p[:, None], (((0,), (0,)), ((), ())),
                         preferred_element_type=_f32)
    gr = gr / jnp.maximum(gc, 1e-6)
    o_ref[...] = jnp.dot(gr, outw_ref[...],
                         preferred_element_type=_f32) + outb_ref[...]


def _tc_epilogue(h, bn, attw, outw, outb):
    return pl.pallas_call(
        _epi_body,
        out_shape=jax.ShapeDtypeStruct((G, D), _f32),
    )(h, bn, attw, outw, outb)


def kernel(x_node, edge_index_ast, edge_index_df, edge_index_cf, batch_node,
           proj_W, proj_b, gcn_W, gcn_b, sgA_Wl, sgA_bl, sgA_Wr,
           sgB_Wl, sgB_bl, sgB_Wr, ln_g, ln_b, att_W, att_b, out_W, out_b):
    (lst0, lst1, lst2, cnt0, cnt1, cnt2,
     deg0, deg1, deg2) = _sc_bucketize(
         edge_index_ast[0], edge_index_ast[1],
         edge_index_df[0], edge_index_df[1],
         edge_index_cf[0], edge_index_cf[1])
    dga = deg0[:N, 0]
    ca = deg1[:N, 0]
    cb = deg2[:N, 0]
    import os as _os
    if _os.environ.get("_BISECT") == "A":
        return (dga[:G, None] + ca[:G, None] + cb[:G, None]
                + jnp.zeros((G, D), _f32)
                + lst0[0, 0] + cnt0[0, 0])

    h, hbf, hpbf, dinv, ia, ib = _tc_prolog(x_node, proj_W, proj_b,
                                            dga, ca, cb)

    perm = [blk * 32 + off
            for blk in range(D // 32)
            for off in list(range(0, 32, 2)) + list(range(1, 32, 2))]

    for l in range(gcn_W.shape[0]):
        u = jnp.concatenate(
            [gcn_W[l][perm, :], sgA_Wl[l][perm, :], sgB_Wl[l][perm, :],
             gcn_W[l], sgA_Wr[l] + sgB_Wr[l]], axis=0)
        bias = gcn_b[l] + sgA_bl[l] + sgB_bl[l]
        hbf32 = lax.bitcast_convert_type(hbf.reshape(N, D // 2, 2), _i32)
        hpbf32 = lax.bitcast_convert_type(hpbf.reshape(N, D // 2, 2), _i32)
        ma, md, mc = _sc_segsum3(hbf32, hpbf32, lst0, lst1, lst2,
                                 cnt0, cnt1, cnt2)
        ma = ma.reshape(NR, D)[:N]
        md = md.reshape(NR, D)[:N]
        mc = mc.reshape(NR, D)[:N]
        h, hbf, hpbf = _tc_layer(ma, md, mc, h, dinv, ia, ib,
                                 u, bias, ln_g, ln_b)

    return _tc_epilogue(h, batch_node, att_W, out_W, out_b)
